# baseline (speedup 1.0000x reference)
import jax
import jax.numpy as jnp
from jax import lax
from jax.experimental import pallas as pl
from jax.experimental.pallas import tpu as pltpu

B = 64
EDD = 2048
EDS = 1024
CHUNK = 256
NCHUNK = EDS // CHUNK
BN_EPS = 1e-5
COS_EPS = 1e-8

_DN_T = (((1,), (1,)), ((), ()))
_DN = (((1,), (0,)), ((), ()))


def _fused_kernel(x_hbm, y_hbm, w_ref, out_ref,
                  xv, yv, dot_acc, nx_acc, ny_acc, sems):
    k = pl.program_id(0)

    @pl.when(k == 0)
    def _():
        cx = pltpu.make_async_copy(x_hbm, xv, sems.at[0])
        cy = pltpu.make_async_copy(y_hbm, yv, sems.at[1])
        cx.start()
        cy.start()
        cx.wait()
        cy.wait()
        dot_acc[...] = jnp.zeros((B, 1), jnp.float32)
        nx_acc[...] = jnp.zeros((B, 1), jnp.float32)
        ny_acc[...] = jnp.zeros((B, 1), jnp.float32)

    ones_row = jnp.ones((1, B), dtype=jnp.float32)
    ones_col = jnp.ones((CHUNK, 1), dtype=jnp.float32)
    lane = lax.broadcasted_iota(jnp.int32, (B, CHUNK), 1)
    at_block_start = (lane % 4) == 0
    low = jnp.full((B, CHUNK), -2.0, dtype=jnp.float32)

    def bn_tanh(hh):
        s1 = lax.dot_general(ones_row, hh, _DN,
                             preferred_element_type=jnp.float32)
        s2 = lax.dot_general(ones_row, hh * hh, _DN,
                             preferred_element_type=jnp.float32)
        mu = s1 * (1.0 / B)
        var = s2 * (1.0 / B) - mu * mu
        scale = lax.rsqrt(var + BN_EPS)
        return jnp.tanh((hh - mu) * scale)

    def block_mask(hh):
        a = jnp.maximum(hh, pltpu.roll(hh, CHUNK - 1, 1))
        bm = jnp.maximum(a, pltpu.roll(a, CHUNK - 2, 1))
        c = jnp.where(at_block_start, bm, low)
        c = jnp.maximum(c, pltpu.roll(c, 1, 1))
        bmax = jnp.maximum(c, pltpu.roll(c, 2, 1))
        return jnp.where(hh == bmax, hh, 0.0)

    w = w_ref[...]
    hx = lax.dot_general(xv[...], w, _DN_T,
                         preferred_element_type=jnp.float32)
    hy = lax.dot_general(yv[...], w, _DN_T,
                         preferred_element_type=jnp.float32)
    mx = block_mask(bn_tanh(hx))
    my = block_mask(bn_tanh(hy))
    dot_acc[...] += lax.dot_general(mx * my, ones_col, _DN,
                                    preferred_element_type=jnp.float32)
    nx_acc[...] += lax.dot_general(mx * mx, ones_col, _DN,
                                   preferred_element_type=jnp.float32)
    ny_acc[...] += lax.dot_general(my * my, ones_col, _DN,
                                   preferred_element_type=jnp.float32)

    @pl.when(k == NCHUNK - 1)
    def _():
        nxc = jnp.maximum(jnp.sqrt(nx_acc[...]), COS_EPS)
        nyc = jnp.maximum(jnp.sqrt(ny_acc[...]), COS_EPS)
        out_ref[...] = (dot_acc[...] / (nxc * nyc)).reshape(B)


def kernel(x, y, W, b, gamma_x, beta_x, gamma_y, beta_y):
    out = pl.pallas_call(
        _fused_kernel,
        grid=(NCHUNK,),
        in_specs=[
            pl.BlockSpec(memory_space=pltpu.MemorySpace.HBM),
            pl.BlockSpec(memory_space=pltpu.MemorySpace.HBM),
            pl.BlockSpec((CHUNK, EDD), lambda k: (k, 0)),
        ],
        out_specs=pl.BlockSpec((B,), lambda k: (0,)),
        out_shape=jax.ShapeDtypeStruct((B,), jnp.float32),
        scratch_shapes=[
            pltpu.VMEM((B, EDD), jnp.float32),
            pltpu.VMEM((B, EDD), jnp.float32),
            pltpu.VMEM((B, 1), jnp.float32),
            pltpu.VMEM((B, 1), jnp.float32),
            pltpu.VMEM((B, 1), jnp.float32),
            pltpu.SemaphoreType.DMA((2,)),
        ],
    )(x, y, W)
    return out


# speedup vs baseline: 1.1199x; 1.1199x over previous
import jax
import jax.numpy as jnp
from jax import lax
from jax.experimental import pallas as pl
from jax.experimental.pallas import tpu as pltpu

B = 64
EDD = 2048
EDS = 1024
CHUNK = 512
NCHUNK = EDS // CHUNK
BN_EPS = 1e-5
COS_EPS = 1e-8

_DN_T = (((1,), (1,)), ((), ()))
_DN = (((1,), (0,)), ((), ()))


def _fused_kernel(x_ref, y_ref, w_hbm, out_ref, wbuf, sems):
    copies = []
    for k in range(NCHUNK):
        c = pltpu.make_async_copy(
            w_hbm.at[pl.ds(k * CHUNK, CHUNK), :], wbuf.at[k], sems.at[k])
        c.start()
        copies.append(c)

    ones_row = jnp.ones((1, B), dtype=jnp.float32)
    ones_col = jnp.ones((CHUNK, 1), dtype=jnp.float32)
    lane = lax.broadcasted_iota(jnp.int32, (B, CHUNK), 1)
    at_block_start = (lane % 4) == 0
    low = jnp.full((B, CHUNK), -2.0, dtype=jnp.float32)

    def bn_tanh(hh):
        s1 = lax.dot_general(ones_row, hh, _DN,
                             preferred_element_type=jnp.float32)
        s2 = lax.dot_general(ones_row, hh * hh, _DN,
                             preferred_element_type=jnp.float32)
        mu = s1 * (1.0 / B)
        var = s2 * (1.0 / B) - mu * mu
        scale = lax.rsqrt(var + BN_EPS)
        return jnp.tanh((hh - mu) * scale)

    def block_mask(hh):
        a = jnp.maximum(hh, pltpu.roll(hh, CHUNK - 1, 1))
        bm = jnp.maximum(a, pltpu.roll(a, CHUNK - 2, 1))
        c = jnp.where(at_block_start, bm, low)
        c = jnp.maximum(c, pltpu.roll(c, 1, 1))
        bmax = jnp.maximum(c, pltpu.roll(c, 2, 1))
        return jnp.where(hh == bmax, hh, 0.0)

    dot = jnp.zeros((B, 1), dtype=jnp.float32)
    nx = jnp.zeros((B, 1), dtype=jnp.float32)
    ny = jnp.zeros((B, 1), dtype=jnp.float32)
    for k in range(NCHUNK):
        copies[k].wait()
        w = wbuf[k]
        hx = lax.dot_general(x_ref[...], w, _DN_T,
                             preferred_element_type=jnp.float32)
        hy = lax.dot_general(y_ref[...], w, _DN_T,
                             preferred_element_type=jnp.float32)
        mx = block_mask(bn_tanh(hx))
        my = block_mask(bn_tanh(hy))
        dot += lax.dot_general(mx * my, ones_col, _DN,
                               preferred_element_type=jnp.float32)
        nx += lax.dot_general(mx * mx, ones_col, _DN,
                              preferred_element_type=jnp.float32)
        ny += lax.dot_general(my * my, ones_col, _DN,
                              preferred_element_type=jnp.float32)

    nxc = jnp.maximum(jnp.sqrt(nx), COS_EPS)
    nyc = jnp.maximum(jnp.sqrt(ny), COS_EPS)
    out_ref[...] = (dot / (nxc * nyc)).reshape(B)


def kernel(x, y, W, b, gamma_x, beta_x, gamma_y, beta_y):
    out = pl.pallas_call(
        _fused_kernel,
        in_specs=[
            pl.BlockSpec((B, EDD), lambda: (0, 0)),
            pl.BlockSpec((B, EDD), lambda: (0, 0)),
            pl.BlockSpec(memory_space=pltpu.MemorySpace.HBM),
        ],
        out_specs=pl.BlockSpec((B,), lambda: (0,)),
        out_shape=jax.ShapeDtypeStruct((B,), jnp.float32),
        scratch_shapes=[
            pltpu.VMEM((NCHUNK, CHUNK, EDD), jnp.float32),
            pltpu.SemaphoreType.DMA((NCHUNK,)),
        ],
    )(x, y, W)
    return out


# speedup vs baseline: 1.3489x; 1.2044x over previous
import jax
import jax.numpy as jnp
from jax import lax
from jax.experimental import pallas as pl
from jax.experimental.pallas import tpu as pltpu

B = 64
B2 = 2 * B
EDD = 2048
EDS = 1024
BN_EPS = 1e-5
COS_EPS = 1e-8

_DN_T = (((1,), (1,)), ((), ()))
_DN = (((1,), (0,)), ((), ()))


def _fused_kernel(x_ref, y_ref, w_ref, out_ref, xy):
    xy[0:B, :] = x_ref[...]
    xy[B:B2, :] = y_ref[...]

    row = lax.broadcasted_iota(jnp.int32, (B2, EDS), 0)
    is_x = row < B
    sel_i = lax.broadcasted_iota(jnp.int32, (2, B2), 0)
    sel_j = lax.broadcasted_iota(jnp.int32, (2, B2), 1)
    sel = jnp.where((sel_j // B) == sel_i, 1.0, 0.0).astype(jnp.float32)

    ones_col = jnp.ones((EDS, 1), dtype=jnp.float32)
    lane = lax.broadcasted_iota(jnp.int32, (B2, EDS), 1)
    at_block_start = (lane % 4) == 0
    low = jnp.full((B2, EDS), -2.0, dtype=jnp.float32)

    w = w_ref[...]
    hh = lax.dot_general(xy[...], w, _DN_T,
                         preferred_element_type=jnp.float32)

    s1 = lax.dot_general(sel, hh, _DN,
                         preferred_element_type=jnp.float32)
    s2 = lax.dot_general(sel, hh * hh, _DN,
                         preferred_element_type=jnp.float32)
    mu2 = s1 * (1.0 / B)
    var2 = s2 * (1.0 / B) - mu2 * mu2
    scale2 = lax.rsqrt(var2 + BN_EPS)
    mu = jnp.where(is_x, mu2[0:1, :], mu2[1:2, :])
    scale = jnp.where(is_x, scale2[0:1, :], scale2[1:2, :])
    th = jnp.tanh((hh - mu) * scale)

    a = jnp.maximum(th, pltpu.roll(th, EDS - 1, 1))
    bm = jnp.maximum(a, pltpu.roll(a, EDS - 2, 1))
    c = jnp.where(at_block_start, bm, low)
    c = jnp.maximum(c, pltpu.roll(c, 1, 1))
    bmax = jnp.maximum(c, pltpu.roll(c, 2, 1))
    m = jnp.where(th == bmax, th, 0.0)

    p = m * pltpu.roll(m, B, 0)
    n = m * m
    P = lax.dot_general(p, ones_col, _DN,
                        preferred_element_type=jnp.float32)
    N = lax.dot_general(n, ones_col, _DN,
                        preferred_element_type=jnp.float32)
    dot = P[0:B, :]
    nxc = jnp.maximum(jnp.sqrt(N[0:B, :]), COS_EPS)
    nyc = jnp.maximum(jnp.sqrt(N[B:B2, :]), COS_EPS)
    out_ref[...] = (dot / (nxc * nyc)).reshape(B)


def kernel(x, y, W, b, gamma_x, beta_x, gamma_y, beta_y):
    out = pl.pallas_call(
        _fused_kernel,
        in_specs=[
            pl.BlockSpec((B, EDD), lambda: (0, 0)),
            pl.BlockSpec((B, EDD), lambda: (0, 0)),
            pl.BlockSpec((EDS, EDD), lambda: (0, 0)),
        ],
        out_specs=pl.BlockSpec((B,), lambda: (0,)),
        out_shape=jax.ShapeDtypeStruct((B,), jnp.float32),
        scratch_shapes=[
            pltpu.VMEM((B2, EDD), jnp.float32),
        ],
    )(x, y, W)
    return out
